# baseline (speedup 1.0000x reference)
import functools

import jax
import jax.numpy as jnp
from jax import lax
from jax.experimental import pallas as pl
from jax.experimental.pallas import tpu as pltpu
from jax.experimental.pallas import tpu_sc as plsc

N_SOURCES = 4096
TOP_K = 64
BATCH = 16384

_NC = 2
_NS = 16
_NW = _NC * _NS
_F = 6144
_RPW = (BATCH - _F) // _NW
_NCHUNKS = 4
_CH = _RPW // _NCHUNKS
_DBLK = 512


def _topk_body(imp_ref, rl_ref, ew_ref, sel_ref, w0_ref, w1_ref):
    imp = imp_ref[:]
    fbits = lax.bitcast_convert_type(imp, jnp.int32)
    skey = jnp.where(fbits >= 0, fbits, fbits ^ jnp.int32(0x7FFFFFFF))
    gidx = (lax.broadcasted_iota(jnp.int32, (32, 128), 0) * 128
            + lax.broadcasted_iota(jnp.int32, (32, 128), 1))
    sign = jnp.int32(-2147483648)

    def bit_step(i, tu):
        cand = tu | (jnp.int32(1) << (31 - i))
        thr = cand ^ sign
        cnt = jnp.sum((skey >= thr).astype(jnp.int32))
        return jnp.where(cnt >= TOP_K, cand, tu)

    tu = lax.fori_loop(0, 32, bit_step, jnp.int32(0))
    kb = tu ^ sign
    m = jnp.sum((skey > kb).astype(jnp.int32))
    r = TOP_K - m

    def tie_step(i, ti):
        cand = ti | (jnp.int32(1) << (12 - i))
        cnt = jnp.sum(((skey == kb) & (gidx < cand)).astype(jnp.int32))
        return jnp.where(cnt <= r, cand, ti)

    ti = lax.fori_loop(0, 13, tie_step, jnp.int32(0))
    sel = (skey > kb) | ((skey == kb) & (gidx < ti))
    self32 = sel.astype(jnp.float32)
    sel_ref[:] = sel.astype(jnp.int32)

    lane = lax.broadcasted_iota(jnp.int32, (128, 128), 0)
    lane_t = lax.broadcasted_iota(jnp.int32, (128, 128), 1)
    tri = (lane <= lane_t).astype(jnp.float32)
    lane_incl = jnp.dot(self32, tri,
                        preferred_element_type=jnp.float32)
    rowsum = jnp.sum(self32, axis=1, keepdims=True)
    rowi = lax.broadcasted_iota(jnp.int32, (32, 32), 0)
    rowj = lax.broadcasted_iota(jnp.int32, (32, 32), 1)
    stri = (rowj < rowi).astype(jnp.float32)
    row_excl = jnp.dot(stri, rowsum,
                       preferred_element_type=jnp.float32)
    rank = (lane_incl + row_excl - 1.0).astype(jnp.int32)

    rl = rl_ref[:]
    mx = jnp.max(rl, axis=1, keepdims=True)
    e = jnp.exp(rl - mx)
    probs = e / jnp.sum(e, axis=1, keepdims=True)

    acc0 = jnp.zeros((32, 128), jnp.float32)
    acc1 = jnp.zeros((32, 128), jnp.float32)
    for k in range(TOP_K):
        hit = rank == k
        acc0 = jnp.where(hit, probs[0, k], acc0)
        acc1 = jnp.where(hit, probs[1, k], acc1)
    ew = ew_ref[:]
    w0_ref[:] = jnp.where(sel, acc0 * ew, 0.0)
    w1_ref[:] = jnp.where(sel, acc1 * ew, 0.0)


def _topk_call(imp2, rl_pad, ew2):
    return pl.pallas_call(
        _topk_body,
        out_shape=[
            jax.ShapeDtypeStruct((32, 128), jnp.int32),
            jax.ShapeDtypeStruct((32, 128), jnp.float32),
            jax.ShapeDtypeStruct((32, 128), jnp.float32),
        ],
    )(imp2, rl_pad, ew2)


def _dense_body(src_ref, w_ref, out_ref):
    out_ref[:] = jnp.dot(src_ref[:], w_ref[:],
                         preferred_element_type=jnp.float32)


def _dense_call(sources, w01):
    return pl.pallas_call(
        _dense_body,
        grid=(_F // _DBLK,),
        in_specs=[
            pl.BlockSpec((_DBLK, N_SOURCES), lambda i: (i, 0)),
            pl.BlockSpec((N_SOURCES, 2), lambda i: (0, 0)),
        ],
        out_specs=pl.BlockSpec((_DBLK, 2), lambda i: (i, 0)),
        out_shape=jax.ShapeDtypeStruct((_F, 2), jnp.float32),
    )(sources, w01)


def _route_body(src_hbm, selm_hbm, w0_hbm, w1_hbm, out0_hbm, out1_hbm,
                selm_v, w0_v, w1_v, idx_buf, c0_buf, c1_buf,
                idxl_a, idxl_b, data_a, data_b,
                out0_v, out1_v, sem_s, sem_a, sem_b):
    wid = lax.axis_index("s") * _NC + lax.axis_index("c")
    base = _F + wid * _RPW

    pltpu.async_copy(selm_hbm, selm_v, sem_s).wait()
    pltpu.async_copy(w0_hbm, w0_v, sem_s).wait()
    pltpu.async_copy(w1_hbm, w1_v, sem_s).wait()

    lane16 = lax.iota(jnp.int32, 16)
    _gdn = lax.GatherDimensionNumbers(
        offset_dims=(), collapsed_slice_dims=(0,), start_index_map=(0,))

    def _gat(v, idx):
        return lax.gather(v, idx[:, None], _gdn, (1,),
                          mode=lax.GatherScatterMode.PROMISE_IN_BOUNDS)

    def comp_step(c, offv):
        mvec = selm_v[pl.ds(c * 16, 16)]
        mb = mvec > 0
        pre = mvec
        for sh in (1, 2, 4, 8):
            shifted = _gat(pre, (lane16 - sh) & 15)
            pre = pre + jnp.where(lane16 >= sh, shifted,
                                  jnp.zeros((16,), jnp.int32))
        posv = offv + pre - 1
        idxvec = c * 16 + lane16
        plsc.store_scatter(idx_buf, [posv], idxvec, mask=mb)
        plsc.store_scatter(c0_buf, [posv], w0_v[pl.ds(c * 16, 16)], mask=mb)
        plsc.store_scatter(c1_buf, [posv], w1_v[pl.ds(c * 16, 16)], mask=mb)
        return offv + _gat(pre, jnp.full((16,), 15, jnp.int32))

    lax.fori_loop(0, N_SOURCES // 16, comp_step,
                  jnp.zeros((16,), jnp.int32))

    idx_chunks = [
        (lambda v: v + (lax.shift_right_logical(v, 7) * 896))(
            idx_buf[pl.ds(j * 16, 16)])
        for j in range(4)
    ]
    c0_chunks = [c0_buf[pl.ds(j * 16, 16)] for j in range(4)]
    c1_chunks = [c1_buf[pl.ds(j * 16, 16)] for j in range(4)]

    def build(t, idxl):
        row0 = base + t * _CH

        def body(r, _):
            rg = row0 + r
            off = (lax.shift_right_logical(rg, 3) * 32768
                   + (rg & 7) * 128)
            for j in range(4):
                idxl[pl.ds(r * 64 + j * 16, 16)] = idx_chunks[j] + off
            return 0

        lax.fori_loop(0, _CH, body, 0)

    def _rotsum(v):
        for sh in (8, 4, 2, 1):
            v = v + _gat(v, (lane16 + sh) & 15)
        return v

    def compute(t, data):
        def body(g, _):
            vec0 = jnp.zeros((16,), jnp.float32)
            vec1 = jnp.zeros((16,), jnp.float32)
            for rr in range(16):
                off = g * (16 * 64) + rr * 64
                acc0 = data[pl.ds(off, 16)] * c0_chunks[0]
                acc1 = data[pl.ds(off, 16)] * c1_chunks[0]
                for j in range(1, 4):
                    d = data[pl.ds(off + j * 16, 16)]
                    acc0 = acc0 + d * c0_chunks[j]
                    acc1 = acc1 + d * c1_chunks[j]
                put = lane16 == rr
                vec0 = jnp.where(put, _rotsum(acc0), vec0)
                vec1 = jnp.where(put, _rotsum(acc1), vec1)
            out0_v[pl.ds(t * _CH + g * 16, 16)] = vec0
            out1_v[pl.ds(t * _CH + g * 16, 16)] = vec1
            return 0

        lax.fori_loop(0, _CH // 16, body, 0)

    bufs = [(idxl_a, data_a, sem_a), (idxl_b, data_b, sem_b)]

    build(0, bufs[0][0])
    copies = {0: pltpu.async_copy(src_hbm.at[bufs[0][0]], bufs[0][1], bufs[0][2])}
    for t in range(_NCHUNKS):
        if t + 1 < _NCHUNKS:
            nb = bufs[(t + 1) % 2]
            build(t + 1, nb[0])
            copies[t + 1] = pltpu.async_copy(src_hbm.at[nb[0]], nb[1], nb[2])
        copies[t].wait()
        compute(t, bufs[t % 2][1])

    obase = wid * _RPW
    pltpu.async_copy(out0_v, out0_hbm.at[pl.ds(obase, _RPW)], sem_s).wait()
    pltpu.async_copy(out1_v, out1_hbm.at[pl.ds(obase, _RPW)], sem_s).wait()


@functools.partial(jax.jit, static_argnums=())
def _route_call(src_flat, selflat, w0f, w1f):
    mesh = plsc.VectorSubcoreMesh(core_axis_name="c", subcore_axis_name="s")
    f = pl.kernel(
        _route_body,
        out_type=[
            jax.ShapeDtypeStruct((BATCH - _F,), jnp.float32),
            jax.ShapeDtypeStruct((BATCH - _F,), jnp.float32),
        ],
        mesh=mesh,
        compiler_params=pltpu.CompilerParams(needs_layout_passes=False),
        scratch_types=[
            pltpu.VMEM((N_SOURCES,), jnp.int32),
            pltpu.VMEM((N_SOURCES,), jnp.float32),
            pltpu.VMEM((N_SOURCES,), jnp.float32),
            pltpu.VMEM((80,), jnp.int32),
            pltpu.VMEM((80,), jnp.float32),
            pltpu.VMEM((80,), jnp.float32),
            pltpu.VMEM((_CH * 64,), jnp.int32),
            pltpu.VMEM((_CH * 64,), jnp.int32),
            pltpu.VMEM((_CH * 64,), jnp.float32),
            pltpu.VMEM((_CH * 64,), jnp.float32),
            pltpu.VMEM((_RPW,), jnp.float32),
            pltpu.VMEM((_RPW,), jnp.float32),
            pltpu.SemaphoreType.DMA,
            pltpu.SemaphoreType.DMA,
            pltpu.SemaphoreType.DMA,
        ],
    )
    return f(src_flat, selflat, w0f, w1f)


def kernel(sources, importance_logits, edge_weights, route_logits):
    imp2 = importance_logits.reshape(32, 128)
    ew2 = edge_weights.reshape(32, 128).astype(jnp.float32)
    rl_pad = jnp.full((8, 128), -1e30, jnp.float32)
    rl_pad = rl_pad.at[:2, :TOP_K].set(route_logits.astype(jnp.float32))

    sel2, w0f, w1f = _topk_call(imp2, rl_pad, ew2)

    src_tiled = sources.reshape(2048, 8, 32, 128).transpose(0, 2, 1, 3)
    sc0, sc1 = _route_call(src_tiled.reshape(-1), sel2.reshape(-1),
                           w0f.reshape(-1), w1f.reshape(-1))

    w01 = jnp.stack([w0f.reshape(-1), w1f.reshape(-1)], axis=1)
    dense = _dense_call(sources, w01)

    out0 = jnp.concatenate([dense[:, 0], sc0])
    out1 = jnp.concatenate([dense[:, 1], sc1])
    return (out0, out1)


# speedup vs baseline: 1.0443x; 1.0443x over previous
import functools

import jax
import jax.numpy as jnp
from jax import lax
from jax.experimental import pallas as pl
from jax.experimental.pallas import tpu as pltpu
from jax.experimental.pallas import tpu_sc as plsc

N_SOURCES = 4096
TOP_K = 64
BATCH = 16384

_NC = 2
_NS = 16
_NW = _NC * _NS
_F = 0
_RPW = (BATCH - _F) // _NW
_NCHUNKS = 4
_CH = _RPW // _NCHUNKS


def _topk_body(imp_ref, rl_ref, ew_ref, sel_ref, w0_ref, w1_ref):
    imp = imp_ref[:]
    fbits = lax.bitcast_convert_type(imp, jnp.int32)
    skey = jnp.where(fbits >= 0, fbits, fbits ^ jnp.int32(0x7FFFFFFF))
    gidx = (lax.broadcasted_iota(jnp.int32, (32, 128), 0) * 128
            + lax.broadcasted_iota(jnp.int32, (32, 128), 1))
    sign = jnp.int32(-2147483648)

    def bit_step(i, tu):
        cand = tu | (jnp.int32(1) << (31 - i))
        thr = cand ^ sign
        cnt = jnp.sum((skey >= thr).astype(jnp.int32))
        return jnp.where(cnt >= TOP_K, cand, tu)

    tu = lax.fori_loop(0, 32, bit_step, jnp.int32(0))
    kb = tu ^ sign
    m = jnp.sum((skey > kb).astype(jnp.int32))
    r = TOP_K - m

    def tie_step(i, ti):
        cand = ti | (jnp.int32(1) << (12 - i))
        cnt = jnp.sum(((skey == kb) & (gidx < cand)).astype(jnp.int32))
        return jnp.where(cnt <= r, cand, ti)

    ti = lax.fori_loop(0, 13, tie_step, jnp.int32(0))
    sel = (skey > kb) | ((skey == kb) & (gidx < ti))
    self32 = sel.astype(jnp.float32)
    sel_ref[:] = sel.astype(jnp.int32)

    lane = lax.broadcasted_iota(jnp.int32, (128, 128), 0)
    lane_t = lax.broadcasted_iota(jnp.int32, (128, 128), 1)
    tri = (lane <= lane_t).astype(jnp.float32)
    lane_incl = jnp.dot(self32, tri,
                        preferred_element_type=jnp.float32)
    rowsum = jnp.sum(self32, axis=1, keepdims=True)
    rowi = lax.broadcasted_iota(jnp.int32, (32, 32), 0)
    rowj = lax.broadcasted_iota(jnp.int32, (32, 32), 1)
    stri = (rowj < rowi).astype(jnp.float32)
    row_excl = jnp.dot(stri, rowsum,
                       preferred_element_type=jnp.float32)
    rank = (lane_incl + row_excl - 1.0).astype(jnp.int32)

    rl = rl_ref[:]
    mx = jnp.max(rl, axis=1, keepdims=True)
    e = jnp.exp(rl - mx)
    probs = e / jnp.sum(e, axis=1, keepdims=True)

    acc0 = jnp.zeros((32, 128), jnp.float32)
    acc1 = jnp.zeros((32, 128), jnp.float32)
    for k in range(TOP_K):
        hit = rank == k
        acc0 = jnp.where(hit, probs[0, k], acc0)
        acc1 = jnp.where(hit, probs[1, k], acc1)
    ew = ew_ref[:]
    w0_ref[:] = jnp.where(sel, acc0 * ew, 0.0)
    w1_ref[:] = jnp.where(sel, acc1 * ew, 0.0)


def _topk_call(imp2, rl_pad, ew2):
    return pl.pallas_call(
        _topk_body,
        out_shape=[
            jax.ShapeDtypeStruct((32, 128), jnp.int32),
            jax.ShapeDtypeStruct((32, 128), jnp.float32),
            jax.ShapeDtypeStruct((32, 128), jnp.float32),
        ],
    )(imp2, rl_pad, ew2)


def _route_body(src_hbm, selm_hbm, w0_hbm, w1_hbm, out0_hbm, out1_hbm,
                selm_v, w0_v, w1_v, idx_buf, c0_buf, c1_buf,
                idxl_a, idxl_b, data_a, data_b,
                out0_v, out1_v, sem_s, sem_a, sem_b):
    wid = lax.axis_index("s") * _NC + lax.axis_index("c")
    base = _F + wid * _RPW

    pltpu.async_copy(selm_hbm, selm_v, sem_s).wait()
    pltpu.async_copy(w0_hbm, w0_v, sem_s).wait()
    pltpu.async_copy(w1_hbm, w1_v, sem_s).wait()

    lane16 = lax.iota(jnp.int32, 16)
    _gdn = lax.GatherDimensionNumbers(
        offset_dims=(), collapsed_slice_dims=(0,), start_index_map=(0,))

    def _gat(v, idx):
        return lax.gather(v, idx[:, None], _gdn, (1,),
                          mode=lax.GatherScatterMode.PROMISE_IN_BOUNDS)

    def comp_step(c, offv):
        mvec = selm_v[pl.ds(c * 16, 16)]
        mb = mvec > 0
        pre = mvec
        for sh in (1, 2, 4, 8):
            shifted = _gat(pre, (lane16 - sh) & 15)
            pre = pre + jnp.where(lane16 >= sh, shifted,
                                  jnp.zeros((16,), jnp.int32))
        posv = offv + pre - 1
        idxvec = c * 16 + lane16
        plsc.store_scatter(idx_buf, [posv], idxvec, mask=mb)
        plsc.store_scatter(c0_buf, [posv], w0_v[pl.ds(c * 16, 16)], mask=mb)
        plsc.store_scatter(c1_buf, [posv], w1_v[pl.ds(c * 16, 16)], mask=mb)
        return offv + _gat(pre, jnp.full((16,), 15, jnp.int32))

    lax.fori_loop(0, N_SOURCES // 16, comp_step,
                  jnp.zeros((16,), jnp.int32))

    idx_chunks = [
        (lambda v: v + (lax.shift_right_logical(v, 7) * 896))(
            idx_buf[pl.ds(j * 16, 16)])
        for j in range(4)
    ]
    c0_chunks = [c0_buf[pl.ds(j * 16, 16)] for j in range(4)]
    c1_chunks = [c1_buf[pl.ds(j * 16, 16)] for j in range(4)]

    def build(t, idxl):
        row0 = base + t * _CH

        def body(r, _):
            rg = row0 + r
            off = (lax.shift_right_logical(rg, 3) * 32768
                   + (rg & 7) * 128)
            for j in range(4):
                idxl[pl.ds(r * 64 + j * 16, 16)] = idx_chunks[j] + off
            return 0

        lax.fori_loop(0, _CH, body, 0)

    def _rotsum(v):
        for sh in (8, 4, 2, 1):
            v = v + _gat(v, (lane16 + sh) & 15)
        return v

    def compute(t, data):
        def body(g, _):
            vec0 = jnp.zeros((16,), jnp.float32)
            vec1 = jnp.zeros((16,), jnp.float32)
            for rr in range(16):
                off = g * (16 * 64) + rr * 64
                acc0 = data[pl.ds(off, 16)] * c0_chunks[0]
                acc1 = data[pl.ds(off, 16)] * c1_chunks[0]
                for j in range(1, 4):
                    d = data[pl.ds(off + j * 16, 16)]
                    acc0 = acc0 + d * c0_chunks[j]
                    acc1 = acc1 + d * c1_chunks[j]
                put = lane16 == rr
                vec0 = jnp.where(put, _rotsum(acc0), vec0)
                vec1 = jnp.where(put, _rotsum(acc1), vec1)
            out0_v[pl.ds(t * _CH + g * 16, 16)] = vec0
            out1_v[pl.ds(t * _CH + g * 16, 16)] = vec1
            return 0

        lax.fori_loop(0, _CH // 16, body, 0)

    bufs = [(idxl_a, data_a, sem_a), (idxl_b, data_b, sem_b)]

    build(0, bufs[0][0])
    copies = {0: pltpu.async_copy(src_hbm.at[bufs[0][0]], bufs[0][1], bufs[0][2])}
    for t in range(_NCHUNKS):
        if t + 1 < _NCHUNKS:
            nb = bufs[(t + 1) % 2]
            build(t + 1, nb[0])
            copies[t + 1] = pltpu.async_copy(src_hbm.at[nb[0]], nb[1], nb[2])
        copies[t].wait()
        compute(t, bufs[t % 2][1])

    obase = wid * _RPW
    pltpu.async_copy(out0_v, out0_hbm.at[pl.ds(obase, _RPW)], sem_s).wait()
    pltpu.async_copy(out1_v, out1_hbm.at[pl.ds(obase, _RPW)], sem_s).wait()


@functools.partial(jax.jit, static_argnums=())
def _route_call(src_flat, selflat, w0f, w1f):
    mesh = plsc.VectorSubcoreMesh(core_axis_name="c", subcore_axis_name="s")
    f = pl.kernel(
        _route_body,
        out_type=[
            jax.ShapeDtypeStruct((BATCH - _F,), jnp.float32),
            jax.ShapeDtypeStruct((BATCH - _F,), jnp.float32),
        ],
        mesh=mesh,
        compiler_params=pltpu.CompilerParams(needs_layout_passes=False),
        scratch_types=[
            pltpu.VMEM((N_SOURCES,), jnp.int32),
            pltpu.VMEM((N_SOURCES,), jnp.float32),
            pltpu.VMEM((N_SOURCES,), jnp.float32),
            pltpu.VMEM((80,), jnp.int32),
            pltpu.VMEM((80,), jnp.float32),
            pltpu.VMEM((80,), jnp.float32),
            pltpu.VMEM((_CH * 64,), jnp.int32),
            pltpu.VMEM((_CH * 64,), jnp.int32),
            pltpu.VMEM((_CH * 64,), jnp.float32),
            pltpu.VMEM((_CH * 64,), jnp.float32),
            pltpu.VMEM((_RPW,), jnp.float32),
            pltpu.VMEM((_RPW,), jnp.float32),
            pltpu.SemaphoreType.DMA,
            pltpu.SemaphoreType.DMA,
            pltpu.SemaphoreType.DMA,
        ],
    )
    return f(src_flat, selflat, w0f, w1f)


def kernel(sources, importance_logits, edge_weights, route_logits):
    imp2 = importance_logits.reshape(32, 128)
    ew2 = edge_weights.reshape(32, 128).astype(jnp.float32)
    rl_pad = jnp.full((8, 128), -1e30, jnp.float32)
    rl_pad = rl_pad.at[:2, :TOP_K].set(route_logits.astype(jnp.float32))

    sel2, w0f, w1f = _topk_call(imp2, rl_pad, ew2)

    src_tiled = sources.reshape(2048, 8, 32, 128).transpose(0, 2, 1, 3)
    out0, out1 = _route_call(src_tiled.reshape(-1), sel2.reshape(-1),
                             w0f.reshape(-1), w1f.reshape(-1))
    return (out0, out1)


# speedup vs baseline: 1.0639x; 1.0188x over previous
import functools

import jax
import jax.numpy as jnp
from jax import lax
from jax.experimental import pallas as pl
from jax.experimental.pallas import tpu as pltpu
from jax.experimental.pallas import tpu_sc as plsc

N_SOURCES = 4096
TOP_K = 64
BATCH = 16384

_NC = 2
_NS = 16
_NW = _NC * _NS
_F = 0
_RPW = (BATCH - _F) // _NW
_NCHUNKS = 4
_CH = _RPW // _NCHUNKS


def _topk_body(imp_ref, rl_ref, ew_ref, sel_ref, w0_ref, w1_ref, meta_ref):
    imp = imp_ref[:]
    fbits = lax.bitcast_convert_type(imp, jnp.int32)
    skey = jnp.where(fbits >= 0, fbits, fbits ^ jnp.int32(0x7FFFFFFF))
    gidx = (lax.broadcasted_iota(jnp.int32, (32, 128), 0) * 128
            + lax.broadcasted_iota(jnp.int32, (32, 128), 1))
    sign = jnp.int32(-2147483648)

    def bit_step(i, tu):
        cand = tu | (jnp.int32(1) << (31 - i))
        thr = cand ^ sign
        cnt = jnp.sum((skey >= thr).astype(jnp.int32))
        return jnp.where(cnt >= TOP_K, cand, tu)

    tu = lax.fori_loop(0, 32, bit_step, jnp.int32(0))
    kb = tu ^ sign
    m = jnp.sum((skey > kb).astype(jnp.int32))
    r = TOP_K - m

    def tie_step(i, ti):
        cand = ti | (jnp.int32(1) << (12 - i))
        cnt = jnp.sum(((skey == kb) & (gidx < cand)).astype(jnp.int32))
        return jnp.where(cnt <= r, cand, ti)

    ti = lax.fori_loop(0, 13, tie_step, jnp.int32(0))
    sel = (skey > kb) | ((skey == kb) & (gidx < ti))
    self32 = sel.astype(jnp.float32)
    sel_ref[:] = sel.astype(jnp.int32)

    lane = lax.broadcasted_iota(jnp.int32, (128, 128), 0)
    lane_t = lax.broadcasted_iota(jnp.int32, (128, 128), 1)
    tri = (lane <= lane_t).astype(jnp.float32)
    lane_incl = jnp.dot(self32, tri,
                        preferred_element_type=jnp.float32)
    rowsum = jnp.sum(self32, axis=1, keepdims=True)
    rowi = lax.broadcasted_iota(jnp.int32, (32, 32), 0)
    rowj = lax.broadcasted_iota(jnp.int32, (32, 32), 1)
    stri = (rowj < rowi).astype(jnp.float32)
    row_excl = jnp.dot(stri, rowsum,
                       preferred_element_type=jnp.float32)
    rank = (lane_incl + row_excl - 1.0).astype(jnp.int32)

    lg = lax.broadcasted_iota(jnp.int32, (128, 8), 0)
    gg = lax.broadcasted_iota(jnp.int32, (128, 8), 1)
    grp = ((lg >> 4) == gg).astype(jnp.float32)
    cs = jnp.dot(self32, grp, preferred_element_type=jnp.float32)
    g1 = lax.broadcasted_iota(jnp.int32, (8, 8), 0)
    g2 = lax.broadcasted_iota(jnp.int32, (8, 8), 1)
    uu = (g1 < g2).astype(jnp.float32)
    coff = jnp.dot(cs, uu, preferred_element_type=jnp.float32) + row_excl
    meta_ref[:] = jnp.concatenate(
        [cs, coff, jnp.zeros((32, 112), jnp.float32)], axis=1)

    rl = rl_ref[:]
    mx = jnp.max(rl, axis=1, keepdims=True)
    e = jnp.exp(rl - mx)
    probs = e / jnp.sum(e, axis=1, keepdims=True)

    acc0 = jnp.zeros((32, 128), jnp.float32)
    acc1 = jnp.zeros((32, 128), jnp.float32)
    for k in range(TOP_K):
        hit = rank == k
        acc0 = jnp.where(hit, probs[0, k], acc0)
        acc1 = jnp.where(hit, probs[1, k], acc1)
    ew = ew_ref[:]
    w0_ref[:] = jnp.where(sel, acc0 * ew, 0.0)
    w1_ref[:] = jnp.where(sel, acc1 * ew, 0.0)


def _topk_call(imp2, rl, ew2):
    return pl.pallas_call(
        _topk_body,
        out_shape=[
            jax.ShapeDtypeStruct((32, 128), jnp.int32),
            jax.ShapeDtypeStruct((32, 128), jnp.float32),
            jax.ShapeDtypeStruct((32, 128), jnp.float32),
            jax.ShapeDtypeStruct((32, 128), jnp.float32),
        ],
    )(imp2, rl, ew2)


def _route_body(src_hbm, selm_hbm, w0_hbm, w1_hbm, meta_hbm,
                out0_hbm, out1_hbm,
                selm_v, w0_v, w1_v, meta_v, idx_buf, c0_buf, c1_buf,
                idxl_a, idxl_b, data_a, data_b,
                out0_v, out1_v, sem_s, sem_a, sem_b):
    wid = lax.axis_index("s") * _NC + lax.axis_index("c")
    base = _F + wid * _RPW

    cps = [pltpu.async_copy(selm_hbm, selm_v, sem_s),
           pltpu.async_copy(w0_hbm, w0_v, sem_s),
           pltpu.async_copy(w1_hbm, w1_v, sem_s),
           pltpu.async_copy(meta_hbm, meta_v, sem_s)]
    for cp in cps:
        cp.wait()

    lane16 = lax.iota(jnp.int32, 16)
    _gdn = lax.GatherDimensionNumbers(
        offset_dims=(), collapsed_slice_dims=(0,), start_index_map=(0,))

    def _gat(v, idx):
        return lax.gather(v, idx[:, None], _gdn, (1,),
                          mode=lax.GatherScatterMode.PROMISE_IN_BOUNDS)

    def comp_row(rr, _):
        meta = meta_v[pl.ds(rr * 128, 16)]
        for g in range(8):
            cnt = jnp.int32(meta[g])
            off = jnp.int32(meta[8 + g])

            @pl.when(cnt > 0)
            def _do(g=g, off=off):
                cbase = rr * 128 + g * 16
                mvec = selm_v[pl.ds(cbase, 16)]
                mb = mvec > 0
                pre = mvec
                for sh in (1, 2, 4, 8):
                    shifted = _gat(pre, (lane16 - sh) & 15)
                    pre = pre + jnp.where(lane16 >= sh, shifted,
                                          jnp.zeros((16,), jnp.int32))
                posv = off + pre - 1
                idxvec = cbase + lane16
                plsc.store_scatter(idx_buf, [posv], idxvec, mask=mb)
                plsc.store_scatter(c0_buf, [posv], w0_v[pl.ds(cbase, 16)],
                                   mask=mb)
                plsc.store_scatter(c1_buf, [posv], w1_v[pl.ds(cbase, 16)],
                                   mask=mb)
        return 0

    lax.fori_loop(0, 32, comp_row, 0)

    idx_chunks = [
        (lambda v: v + (lax.shift_right_logical(v, 7) * 896))(
            idx_buf[pl.ds(j * 16, 16)])
        for j in range(4)
    ]
    c0_chunks = [c0_buf[pl.ds(j * 16, 16)] for j in range(4)]
    c1_chunks = [c1_buf[pl.ds(j * 16, 16)] for j in range(4)]

    def build(t, idxl):
        row0 = base + t * _CH

        def body(r, _):
            rg = row0 + r
            off = (lax.shift_right_logical(rg, 3) * 32768
                   + (rg & 7) * 128)
            for j in range(4):
                idxl[pl.ds(r * 64 + j * 16, 16)] = idx_chunks[j] + off
            return 0

        lax.fori_loop(0, _CH, body, 0)

    def _rotsum(v):
        for sh in (8, 4, 2, 1):
            v = v + _gat(v, (lane16 + sh) & 15)
        return v

    def compute(t, data):
        def body(g, _):
            vec0 = jnp.zeros((16,), jnp.float32)
            vec1 = jnp.zeros((16,), jnp.float32)
            for rr in range(16):
                off = g * (16 * 64) + rr * 64
                acc0 = data[pl.ds(off, 16)] * c0_chunks[0]
                acc1 = data[pl.ds(off, 16)] * c1_chunks[0]
                for j in range(1, 4):
                    d = data[pl.ds(off + j * 16, 16)]
                    acc0 = acc0 + d * c0_chunks[j]
                    acc1 = acc1 + d * c1_chunks[j]
                put = lane16 == rr
                vec0 = jnp.where(put, _rotsum(acc0), vec0)
                vec1 = jnp.where(put, _rotsum(acc1), vec1)
            out0_v[pl.ds(t * _CH + g * 16, 16)] = vec0
            out1_v[pl.ds(t * _CH + g * 16, 16)] = vec1
            return 0

        lax.fori_loop(0, _CH // 16, body, 0)

    bufs = [(idxl_a, data_a, sem_a), (idxl_b, data_b, sem_b)]

    build(0, bufs[0][0])
    copies = {0: pltpu.async_copy(src_hbm.at[bufs[0][0]], bufs[0][1], bufs[0][2])}
    for t in range(_NCHUNKS):
        if t + 1 < _NCHUNKS:
            nb = bufs[(t + 1) % 2]
            build(t + 1, nb[0])
            copies[t + 1] = pltpu.async_copy(src_hbm.at[nb[0]], nb[1], nb[2])
        copies[t].wait()
        compute(t, bufs[t % 2][1])

    obase = wid * _RPW
    pltpu.async_copy(out0_v, out0_hbm.at[pl.ds(obase, _RPW)], sem_s).wait()
    pltpu.async_copy(out1_v, out1_hbm.at[pl.ds(obase, _RPW)], sem_s).wait()


@functools.partial(jax.jit, static_argnums=())
def _route_call(src_flat, selflat, w0f, w1f, metaf):
    mesh = plsc.VectorSubcoreMesh(core_axis_name="c", subcore_axis_name="s")
    f = pl.kernel(
        _route_body,
        out_type=[
            jax.ShapeDtypeStruct((BATCH - _F,), jnp.float32),
            jax.ShapeDtypeStruct((BATCH - _F,), jnp.float32),
        ],
        mesh=mesh,
        compiler_params=pltpu.CompilerParams(needs_layout_passes=False),
        scratch_types=[
            pltpu.VMEM((N_SOURCES,), jnp.int32),
            pltpu.VMEM((N_SOURCES,), jnp.float32),
            pltpu.VMEM((N_SOURCES,), jnp.float32),
            pltpu.VMEM((N_SOURCES,), jnp.float32),
            pltpu.VMEM((80,), jnp.int32),
            pltpu.VMEM((80,), jnp.float32),
            pltpu.VMEM((80,), jnp.float32),
            pltpu.VMEM((_CH * 64,), jnp.int32),
            pltpu.VMEM((_CH * 64,), jnp.int32),
            pltpu.VMEM((_CH * 64,), jnp.float32),
            pltpu.VMEM((_CH * 64,), jnp.float32),
            pltpu.VMEM((_RPW,), jnp.float32),
            pltpu.VMEM((_RPW,), jnp.float32),
            pltpu.SemaphoreType.DMA,
            pltpu.SemaphoreType.DMA,
            pltpu.SemaphoreType.DMA,
        ],
    )
    return f(src_flat, selflat, w0f, w1f, metaf)


def kernel(sources, importance_logits, edge_weights, route_logits):
    imp2 = importance_logits.reshape(32, 128)
    ew2 = edge_weights.reshape(32, 128).astype(jnp.float32)

    sel2, w0f, w1f, meta = _topk_call(
        imp2, route_logits.astype(jnp.float32), ew2)

    src_tiled = sources.reshape(2048, 8, 32, 128).transpose(0, 2, 1, 3)
    out0, out1 = _route_call(src_tiled.reshape(-1), sel2.reshape(-1),
                             w0f.reshape(-1), w1f.reshape(-1),
                             meta.reshape(-1))
    return (out0, out1)


# speedup vs baseline: 1.0639x; 1.0000x over previous
import functools

import jax
import jax.numpy as jnp
from jax import lax
from jax.experimental import pallas as pl
from jax.experimental.pallas import tpu as pltpu
from jax.experimental.pallas import tpu_sc as plsc

N_SOURCES = 4096
TOP_K = 64
BATCH = 16384

_NC = 2
_NS = 16
_NW = _NC * _NS
_F = 0
_RPW = (BATCH - _F) // _NW
_NCHUNKS = 8
_CH = _RPW // _NCHUNKS


def _topk_body(imp_ref, rl_ref, ew_ref, sel_ref, w0_ref, w1_ref, meta_ref):
    imp = imp_ref[:]
    fbits = lax.bitcast_convert_type(imp, jnp.int32)
    skey = jnp.where(fbits >= 0, fbits, fbits ^ jnp.int32(0x7FFFFFFF))
    gidx = (lax.broadcasted_iota(jnp.int32, (32, 128), 0) * 128
            + lax.broadcasted_iota(jnp.int32, (32, 128), 1))
    sign = jnp.int32(-2147483648)

    def bit_step(i, tu):
        cand = tu | (jnp.int32(1) << (31 - i))
        thr = cand ^ sign
        cnt = jnp.sum((skey >= thr).astype(jnp.int32))
        return jnp.where(cnt >= TOP_K, cand, tu)

    tu = lax.fori_loop(0, 32, bit_step, jnp.int32(0))
    kb = tu ^ sign
    m = jnp.sum((skey > kb).astype(jnp.int32))
    r = TOP_K - m

    def tie_step(i, ti):
        cand = ti | (jnp.int32(1) << (12 - i))
        cnt = jnp.sum(((skey == kb) & (gidx < cand)).astype(jnp.int32))
        return jnp.where(cnt <= r, cand, ti)

    ti = lax.fori_loop(0, 13, tie_step, jnp.int32(0))
    sel = (skey > kb) | ((skey == kb) & (gidx < ti))
    self32 = sel.astype(jnp.float32)
    sel_ref[:] = sel.astype(jnp.int32)

    lane = lax.broadcasted_iota(jnp.int32, (128, 128), 0)
    lane_t = lax.broadcasted_iota(jnp.int32, (128, 128), 1)
    tri = (lane <= lane_t).astype(jnp.float32)
    lane_incl = jnp.dot(self32, tri,
                        preferred_element_type=jnp.float32)
    rowsum = jnp.sum(self32, axis=1, keepdims=True)
    rowi = lax.broadcasted_iota(jnp.int32, (32, 32), 0)
    rowj = lax.broadcasted_iota(jnp.int32, (32, 32), 1)
    stri = (rowj < rowi).astype(jnp.float32)
    row_excl = jnp.dot(stri, rowsum,
                       preferred_element_type=jnp.float32)
    rank = (lane_incl + row_excl - 1.0).astype(jnp.int32)

    lg = lax.broadcasted_iota(jnp.int32, (128, 8), 0)
    gg = lax.broadcasted_iota(jnp.int32, (128, 8), 1)
    grp = ((lg >> 4) == gg).astype(jnp.float32)
    cs = jnp.dot(self32, grp, preferred_element_type=jnp.float32)
    g1 = lax.broadcasted_iota(jnp.int32, (8, 8), 0)
    g2 = lax.broadcasted_iota(jnp.int32, (8, 8), 1)
    uu = (g1 < g2).astype(jnp.float32)
    coff = jnp.dot(cs, uu, preferred_element_type=jnp.float32) + row_excl
    meta_ref[:] = jnp.concatenate(
        [cs, coff, jnp.zeros((32, 112), jnp.float32)], axis=1)

    rl = rl_ref[:]
    mx = jnp.max(rl, axis=1, keepdims=True)
    e = jnp.exp(rl - mx)
    probs = e / jnp.sum(e, axis=1, keepdims=True)

    acc0 = jnp.zeros((32, 128), jnp.float32)
    acc1 = jnp.zeros((32, 128), jnp.float32)
    for k in range(TOP_K):
        hit = rank == k
        acc0 = jnp.where(hit, probs[0, k], acc0)
        acc1 = jnp.where(hit, probs[1, k], acc1)
    ew = ew_ref[:]
    w0_ref[:] = jnp.where(sel, acc0 * ew, 0.0)
    w1_ref[:] = jnp.where(sel, acc1 * ew, 0.0)


def _topk_call(imp2, rl, ew2):
    return pl.pallas_call(
        _topk_body,
        out_shape=[
            jax.ShapeDtypeStruct((32, 128), jnp.int32),
            jax.ShapeDtypeStruct((32, 128), jnp.float32),
            jax.ShapeDtypeStruct((32, 128), jnp.float32),
            jax.ShapeDtypeStruct((32, 128), jnp.float32),
        ],
    )(imp2, rl, ew2)


def _route_body(src_hbm, selm_hbm, w0_hbm, w1_hbm, meta_hbm,
                out0_hbm, out1_hbm,
                selm_v, w0_v, w1_v, meta_v, idx_buf, c0_buf, c1_buf,
                idxl_a, idxl_b, data_a, data_b,
                out0_v, out1_v, sem_s, sem_a, sem_b):
    wid = lax.axis_index("s") * _NC + lax.axis_index("c")
    base = _F + wid * _RPW

    cps = [pltpu.async_copy(selm_hbm, selm_v, sem_s),
           pltpu.async_copy(w0_hbm, w0_v, sem_s),
           pltpu.async_copy(w1_hbm, w1_v, sem_s),
           pltpu.async_copy(meta_hbm, meta_v, sem_s)]
    for cp in cps:
        cp.wait()

    lane16 = lax.iota(jnp.int32, 16)
    _gdn = lax.GatherDimensionNumbers(
        offset_dims=(), collapsed_slice_dims=(0,), start_index_map=(0,))

    def _gat(v, idx):
        return lax.gather(v, idx[:, None], _gdn, (1,),
                          mode=lax.GatherScatterMode.PROMISE_IN_BOUNDS)

    def comp_row(rr, _):
        meta = meta_v[pl.ds(rr * 128, 16)]
        for g in range(8):
            cnt = jnp.int32(meta[g])
            off = jnp.int32(meta[8 + g])

            @pl.when(cnt > 0)
            def _do(g=g, off=off):
                cbase = rr * 128 + g * 16
                mvec = selm_v[pl.ds(cbase, 16)]
                mb = mvec > 0
                pre = mvec
                for sh in (1, 2, 4, 8):
                    shifted = _gat(pre, (lane16 - sh) & 15)
                    pre = pre + jnp.where(lane16 >= sh, shifted,
                                          jnp.zeros((16,), jnp.int32))
                posv = off + pre - 1
                idxvec = cbase + lane16
                plsc.store_scatter(idx_buf, [posv], idxvec, mask=mb)
                plsc.store_scatter(c0_buf, [posv], w0_v[pl.ds(cbase, 16)],
                                   mask=mb)
                plsc.store_scatter(c1_buf, [posv], w1_v[pl.ds(cbase, 16)],
                                   mask=mb)
        return 0

    lax.fori_loop(0, 32, comp_row, 0)

    idx_chunks = [
        (lambda v: v + (lax.shift_right_logical(v, 7) * 896))(
            idx_buf[pl.ds(j * 16, 16)])
        for j in range(4)
    ]
    c0_chunks = [c0_buf[pl.ds(j * 16, 16)] for j in range(4)]
    c1_chunks = [c1_buf[pl.ds(j * 16, 16)] for j in range(4)]

    def build(t, idxl):
        row0 = base + t * _CH

        def body(r, _):
            rg = row0 + r
            off = (lax.shift_right_logical(rg, 3) * 32768
                   + (rg & 7) * 128)
            for j in range(4):
                idxl[pl.ds(r * 64 + j * 16, 16)] = idx_chunks[j] + off
            return 0

        lax.fori_loop(0, _CH, body, 0)

    def _rotsum(v):
        for sh in (8, 4, 2, 1):
            v = v + _gat(v, (lane16 + sh) & 15)
        return v

    def compute(t, data):
        def body(g, _):
            vec0 = jnp.zeros((16,), jnp.float32)
            vec1 = jnp.zeros((16,), jnp.float32)
            for rr in range(16):
                off = g * (16 * 64) + rr * 64
                acc0 = data[pl.ds(off, 16)] * c0_chunks[0]
                acc1 = data[pl.ds(off, 16)] * c1_chunks[0]
                for j in range(1, 4):
                    d = data[pl.ds(off + j * 16, 16)]
                    acc0 = acc0 + d * c0_chunks[j]
                    acc1 = acc1 + d * c1_chunks[j]
                put = lane16 == rr
                vec0 = jnp.where(put, _rotsum(acc0), vec0)
                vec1 = jnp.where(put, _rotsum(acc1), vec1)
            out0_v[pl.ds(t * _CH + g * 16, 16)] = vec0
            out1_v[pl.ds(t * _CH + g * 16, 16)] = vec1
            return 0

        lax.fori_loop(0, _CH // 16, body, 0)

    bufs = [(idxl_a, data_a, sem_a), (idxl_b, data_b, sem_b)]

    build(0, bufs[0][0])
    copies = {0: pltpu.async_copy(src_hbm.at[bufs[0][0]], bufs[0][1], bufs[0][2])}
    for t in range(_NCHUNKS):
        if t + 1 < _NCHUNKS:
            nb = bufs[(t + 1) % 2]
            build(t + 1, nb[0])
            copies[t + 1] = pltpu.async_copy(src_hbm.at[nb[0]], nb[1], nb[2])
        copies[t].wait()
        compute(t, bufs[t % 2][1])

    obase = wid * _RPW
    pltpu.async_copy(out0_v, out0_hbm.at[pl.ds(obase, _RPW)], sem_s).wait()
    pltpu.async_copy(out1_v, out1_hbm.at[pl.ds(obase, _RPW)], sem_s).wait()


@functools.partial(jax.jit, static_argnums=())
def _route_call(src_flat, selflat, w0f, w1f, metaf):
    mesh = plsc.VectorSubcoreMesh(core_axis_name="c", subcore_axis_name="s")
    f = pl.kernel(
        _route_body,
        out_type=[
            jax.ShapeDtypeStruct((BATCH - _F,), jnp.float32),
            jax.ShapeDtypeStruct((BATCH - _F,), jnp.float32),
        ],
        mesh=mesh,
        compiler_params=pltpu.CompilerParams(needs_layout_passes=False),
        scratch_types=[
            pltpu.VMEM((N_SOURCES,), jnp.int32),
            pltpu.VMEM((N_SOURCES,), jnp.float32),
            pltpu.VMEM((N_SOURCES,), jnp.float32),
            pltpu.VMEM((N_SOURCES,), jnp.float32),
            pltpu.VMEM((80,), jnp.int32),
            pltpu.VMEM((80,), jnp.float32),
            pltpu.VMEM((80,), jnp.float32),
            pltpu.VMEM((_CH * 64,), jnp.int32),
            pltpu.VMEM((_CH * 64,), jnp.int32),
            pltpu.VMEM((_CH * 64,), jnp.float32),
            pltpu.VMEM((_CH * 64,), jnp.float32),
            pltpu.VMEM((_RPW,), jnp.float32),
            pltpu.VMEM((_RPW,), jnp.float32),
            pltpu.SemaphoreType.DMA,
            pltpu.SemaphoreType.DMA,
            pltpu.SemaphoreType.DMA,
        ],
    )
    return f(src_flat, selflat, w0f, w1f, metaf)


def kernel(sources, importance_logits, edge_weights, route_logits):
    imp2 = importance_logits.reshape(32, 128)
    ew2 = edge_weights.reshape(32, 128).astype(jnp.float32)

    sel2, w0f, w1f, meta = _topk_call(
        imp2, route_logits.astype(jnp.float32), ew2)

    src_tiled = sources.reshape(2048, 8, 32, 128).transpose(0, 2, 1, 3)
    out0, out1 = _route_call(src_tiled.reshape(-1), sel2.reshape(-1),
                             w0f.reshape(-1), w1f.reshape(-1),
                             meta.reshape(-1))
    return (out0, out1)
